# baseline (speedup 1.0000x reference)
import functools

import jax
import jax.numpy as jnp
from jax import lax
from jax.experimental import pallas as pl
from jax.experimental.pallas import tpu as pltpu
from jax.experimental.pallas import tpu_sc as plsc

_N, _NPAD, _F, _HH, _H = 10000, 10240, 128, 48, 96
_NC, _NS, _NW, _CH = 2, 16, 32, 128
_C1H = 80
_C1 = 160
_C2 = 316
_C2H = 158
_CW = 2
_EC = _CW * 128
_PADROW = _NPAD - 1
_STR = _NPAD // _NS
_BN = 1280
_HIGH = lax.Precision.HIGHEST

_mesh = plsc.VectorSubcoreMesh(core_axis_name="c", subcore_axis_name="s")
_sc_params = pltpu.CompilerParams(use_tc_tiling_on_sc=False)


def _hist_scatter(ones_v, acc, idx_v, semA, semB, drain, nh):
    @pl.loop(0, nh, step=2)
    def _(j):
        @pl.when(j > 0)
        def _():
            pltpu.make_async_copy(drain, ones_v, semA).wait()
            pltpu.make_async_copy(drain, ones_v, semB).wait()
        pltpu.async_copy(ones_v, acc.at[idx_v.at[j]], semA, add=True)
        pltpu.async_copy(ones_v, acc.at[idx_v.at[j + 1]], semB, add=True)
    pltpu.make_async_copy(drain, ones_v, semA).wait()
    pltpu.make_async_copy(drain, ones_v, semB).wait()


@functools.partial(
    pl.kernel,
    out_type=jax.ShapeDtypeStruct((_NC, 2, _NPAD, 16), jnp.float32),
    mesh=_mesh,
    compiler_params=_sc_params,
    scratch_types=[
        pltpu.VMEM_SHARED((_NPAD, 16), jnp.float32),
        pltpu.VMEM_SHARED((_NPAD, 16), jnp.float32),
        pltpu.VMEM((_EC, 16), jnp.float32),
        pltpu.VMEM((_C1H // _CW, _EC), jnp.int32),
        pltpu.SemaphoreType.DMA,
        pltpu.SemaphoreType.DMA,
    ],
)
def _sc_hist(cc_hbm, cv_hbm, ones_hbm, zeros_hbm, out_hbm,
             acc_c, acc_v, ones_v, idx_v, semA, semB):
    cid = lax.axis_index("c")
    sid = lax.axis_index("s")
    base = sid * _STR
    pltpu.sync_copy(zeros_hbm, acc_c.at[pl.ds(base, _STR)])
    pltpu.sync_copy(zeros_hbm, acc_v.at[pl.ds(base, _STR)])
    pltpu.sync_copy(ones_hbm, ones_v)
    plsc.subcore_barrier()

    nh = _C1H // _CW
    drain = zeros_hbm.at[pl.ds(0, _EC)]
    pltpu.sync_copy(cc_hbm.at[sid, pl.ds(cid * nh, nh)], idx_v)
    _hist_scatter(ones_v, acc_c, idx_v, semA, semB, drain, nh)
    pltpu.sync_copy(cv_hbm.at[sid, pl.ds(cid * nh, nh)], idx_v)
    _hist_scatter(ones_v, acc_v, idx_v, semA, semB, drain, nh)

    plsc.subcore_barrier()
    pltpu.sync_copy(acc_c.at[pl.ds(base, _STR)],
                    out_hbm.at[cid, 0, pl.ds(base, _STR)])
    pltpu.sync_copy(acc_v.at[pl.ds(base, _STR)],
                    out_hbm.at[cid, 1, pl.ds(base, _STR)])


def _agg_edges(table, idxr_v, idxc_v, acc, gbs, semg, sems, drain_src, n_chunks):
    nb = len(gbs)
    for b in range(nb - 1):
        pltpu.async_copy(table.at[idxr_v.at[b]], gbs[b], semg[b])

    @pl.loop(0, n_chunks, step=nb)
    def _(j):
        for b in range(nb):
            jj = j + b
            nxt = jj + (nb - 1)
            bb = (b + nb - 1) % nb
            pltpu.make_async_copy(
                table.at[idxr_v.at[jj]], gbs[b], semg[b]).wait()
            pltpu.async_copy(
                gbs[b], acc.at[idxc_v.at[jj]], sems[b], add=True)

            @pl.when(nxt < n_chunks)
            def _():
                @pl.when(nxt >= nb)
                def _():
                    pltpu.make_async_copy(drain_src, gbs[bb], sems[bb]).wait()

                pltpu.async_copy(
                    table.at[idxr_v.at[nxt]], gbs[bb], semg[bb])

    for b in range(nb):
        pltpu.make_async_copy(drain_src, gbs[b], sems[b]).wait()


_AGG_SCRATCH = [
    pltpu.VMEM_SHARED((_NPAD, _HH), jnp.float32),
    pltpu.VMEM_SHARED((_NPAD, _HH), jnp.float32),
    pltpu.VMEM((_C1 // _CW, _EC), jnp.int32),
    pltpu.VMEM((_C1 // _CW, _EC), jnp.int32),
    pltpu.VMEM((_EC, _HH), jnp.float32),
    pltpu.VMEM((_EC, _HH), jnp.float32),
    pltpu.SemaphoreType.DMA,
    pltpu.SemaphoreType.DMA,
    pltpu.SemaphoreType.DMA,
    pltpu.SemaphoreType.DMA,
]


@functools.partial(
    pl.kernel,
    out_type=jax.ShapeDtypeStruct((_NC, _NPAD, _HH), jnp.float32),
    mesh=_mesh,
    compiler_params=_sc_params,
    scratch_types=_AGG_SCRATCH,
)
def _sc_agg1(hsc_hbm, hsv_hbm, rc_hbm, cc_hbm, rv_hbm, cv_hbm, zeros_hbm,
             out_hbm, acc, table, idxr_v, idxc_v,
             gb0, gb1, sg0, sg1, ss0, ss1):
    cid = lax.axis_index("c")
    sid = lax.axis_index("s")
    base = sid * _STR
    pltpu.sync_copy(zeros_hbm, acc.at[pl.ds(base, _STR)])

    @pl.when(cid == 0)
    def _():
        pltpu.sync_copy(hsc_hbm.at[pl.ds(base, _STR)], table.at[pl.ds(base, _STR)])
        pltpu.sync_copy(rc_hbm.at[sid], idxr_v)
        pltpu.sync_copy(cc_hbm.at[sid], idxc_v)

    @pl.when(cid == 1)
    def _():
        pltpu.sync_copy(hsv_hbm.at[pl.ds(base, _STR)], table.at[pl.ds(base, _STR)])
        pltpu.sync_copy(rv_hbm.at[sid], idxr_v)
        pltpu.sync_copy(cv_hbm.at[sid], idxc_v)

    plsc.subcore_barrier()
    _agg_edges(table, idxr_v, idxc_v, acc, (gb0, gb1),
               (sg0, sg1), (ss0, ss1),
               zeros_hbm.at[pl.ds(0, _EC)], _C1 // _CW)
    plsc.subcore_barrier()
    pltpu.sync_copy(acc.at[pl.ds(base, _STR)],
                    out_hbm.at[cid, pl.ds(base, _STR)])


@functools.partial(
    pl.kernel,
    out_type=jax.ShapeDtypeStruct((_NC, _NPAD, _HH), jnp.float32),
    mesh=_mesh,
    compiler_params=_sc_params,
    scratch_types=_AGG_SCRATCH,
)
def _sc_agg2(hs2a_hbm, hs2b_hbm, rc_hbm, cc_hbm, rv_hbm, cv_hbm, zeros_hbm,
             out_hbm, acc, table, idxr_v, idxc_v,
             gb0, gb1, sg0, sg1, ss0, ss1):
    cid = lax.axis_index("c")
    sid = lax.axis_index("s")
    base = sid * _STR
    pltpu.sync_copy(zeros_hbm, acc.at[pl.ds(base, _STR)])

    @pl.when(cid == 0)
    def _():
        pltpu.sync_copy(hs2a_hbm.at[pl.ds(base, _STR)], table.at[pl.ds(base, _STR)])

    @pl.when(cid == 1)
    def _():
        pltpu.sync_copy(hs2b_hbm.at[pl.ds(base, _STR)], table.at[pl.ds(base, _STR)])

    plsc.subcore_barrier()

    for r_hbm, c_hbm in ((rc_hbm, cc_hbm), (rv_hbm, cv_hbm)):
        pltpu.sync_copy(r_hbm.at[sid], idxr_v)
        pltpu.sync_copy(c_hbm.at[sid], idxc_v)
        _agg_edges(table, idxr_v, idxc_v, acc, (gb0, gb1),
                   (sg0, sg1), (ss0, ss1),
                   zeros_hbm.at[pl.ds(0, _EC)], _C1 // _CW)

    plsc.subcore_barrier()
    pltpu.sync_copy(acc.at[pl.ds(base, _STR)],
                    out_hbm.at[cid, pl.ds(base, _STR)])


def _ln_elu(v, g, b):
    m = jnp.mean(v, axis=-1, keepdims=True)
    var = jnp.mean((v - m) ** 2, axis=-1, keepdims=True)
    u = (v - m) / jnp.sqrt(var + 1e-5) * g + b
    return jnp.where(u > 0, u, jnp.exp(jnp.minimum(u, 0.0)) - 1.0)


def _deg_scales(cnt):
    cntc = cnt[0, 0, :, 0] + cnt[1, 0, :, 0]
    cntv = cnt[0, 1, :, 0] + cnt[1, 1, :, 0]
    disc = lax.rsqrt(cntc + 1.0)
    disv = lax.rsqrt(cntv + 1.0)
    dis2 = lax.rsqrt(cntc + cntv + 1.0)
    return disc, disv, dis2


def _tc1_body(cnt_ref, x_ref, xl_ref, wc_ref, wva_ref, wvb_ref,
              hsc_ref, hsv_ref):
    disc, disv, _ = _deg_scales(cnt_ref[...])
    xb = x_ref[...]
    hc = jnp.dot(xb, wc_ref[...], precision=_HIGH)
    hv = (jnp.dot(xb, wva_ref[...], precision=_HIGH)
          + jnp.dot(xl_ref[...], wvb_ref[...], precision=_HIGH))
    row = _BN * pl.program_id(0) + lax.broadcasted_iota(jnp.int32, (_BN, 1), 0)
    live = row < _N
    hsc_ref[...] = jnp.where(live, hc * disc[:, None], 0.0)
    hsv_ref[...] = jnp.where(live, hv * disv[:, None], 0.0)


def _tc2_body(agg_ref, hsc_ref, hsv_ref, cnt_ref, wra_ref, wrb_ref,
              pc_ref, pv_ref, hs2a_ref, hs2b_ref):
    disc, disv, dis2 = _deg_scales(cnt_ref[...])
    pc = pc_ref[...]
    pv = pv_ref[...]
    oc = disc[:, None] * (agg_ref[0] + hsc_ref[...]) + pc[0]
    ov = disv[:, None] * (agg_ref[1] + hsv_ref[...]) + pv[0]
    uc = _ln_elu(oc, pc[1], pc[2])
    uv = _ln_elu(ov, pv[1], pv[2])
    h2 = (jnp.dot(uc, wra_ref[...], precision=_HIGH)
          + jnp.dot(uv, wrb_ref[...], precision=_HIGH))
    hs2 = h2 * dis2[:, None]
    hs2a_ref[...] = hs2[:, :_HH]
    hs2b_ref[...] = hs2[:, _HH:]


def _tc3_body(agg2_ref, hs2a_ref, hs2b_ref, cnt_ref, pr_ref, tail_ref, out_ref):
    _, _, dis2 = _deg_scales(cnt_ref[...])
    pr = pr_ref[...]
    agg2 = jnp.concatenate([agg2_ref[0], agg2_ref[1]], axis=1)
    hs2 = jnp.concatenate([hs2a_ref[...], hs2b_ref[...]], axis=1)
    o = dis2[:, None] * (agg2 + hs2) + pr[0]
    u = _ln_elu(o, pr[1], pr[2])
    tail = tail_ref[...]
    wo = tail[0, :_H]
    bo = tail[0, _H]
    out_ref[...] = (jnp.sum(u * wo[None, :], axis=1) + bo)[:, None]


def _full(shape):
    return pl.BlockSpec(shape, lambda i: tuple(0 for _ in shape))


def _rows(w):
    return pl.BlockSpec((_BN, w), lambda i: (i, 0))


_GRID = (_NPAD // _BN,)
_CNT_SPEC = pl.BlockSpec((_NC, 2, _BN, 16), lambda i: (0, 0, i, 0))

_tc1 = pl.pallas_call(
    _tc1_body,
    grid=_GRID,
    in_specs=[_CNT_SPEC, _rows(_F), _rows(_F),
              _full((_F, _HH)), _full((_F, _HH)), _full((_F, _HH))],
    out_specs=(_rows(_HH), _rows(_HH)),
    out_shape=(jax.ShapeDtypeStruct((_NPAD, _HH), jnp.float32),
               jax.ShapeDtypeStruct((_NPAD, _HH), jnp.float32)),
)

_tc2 = pl.pallas_call(
    _tc2_body,
    grid=_GRID,
    in_specs=[pl.BlockSpec((_NC, _BN, _HH), lambda i: (0, i, 0)),
              _rows(_HH), _rows(_HH), _CNT_SPEC,
              _full((_HH, _H)), _full((_HH, _H)),
              _full((3, _HH)), _full((3, _HH))],
    out_specs=(_rows(_HH), _rows(_HH)),
    out_shape=(jax.ShapeDtypeStruct((_NPAD, _HH), jnp.float32),
               jax.ShapeDtypeStruct((_NPAD, _HH), jnp.float32)),
)

_tc3 = pl.pallas_call(
    _tc3_body,
    grid=_GRID,
    in_specs=[pl.BlockSpec((_NC, _BN, _HH), lambda i: (0, i, 0)),
              _rows(_HH), _rows(_HH), _CNT_SPEC,
              _full((3, _H)), _full((1, _F))],
    out_specs=pl.BlockSpec((_BN, 1), lambda i: (i, 0)),
    out_shape=jax.ShapeDtypeStruct((_NPAD, 1), jnp.float32),
)


def _prep_idx(idx, n_parts, n_chunks):
    e = idx.shape[0]
    epad = n_parts * n_chunks * _EC
    p = jnp.full((epad,), _PADROW, jnp.int32).at[:e].set(idx)
    return p.reshape(n_parts, n_chunks, _EC)


def kernel(x, edge_index_corr, edge_index_vendor, x_lagged,
           W_corr, b_corr, g_corr, beta_corr,
           W_vendor, b_vendor, g_vendor, beta_vendor,
           W_refine, b_refine, g_refine, beta_refine,
           W_out, b_out):
    f32 = jnp.float32
    rc = _prep_idx(edge_index_corr[0], _NS, _C1 // _CW)
    cc = _prep_idx(edge_index_corr[1], _NS, _C1 // _CW)
    rv = _prep_idx(edge_index_vendor[0], _NS, _C1 // _CW)
    cv = _prep_idx(edge_index_vendor[1], _NS, _C1 // _CW)

    ones16 = jnp.ones((_EC, 16), f32)
    z16 = jnp.zeros((_STR, 16), f32)
    z48 = jnp.zeros((_STR, _HH), f32)

    wc_t = W_corr.T
    wva_t = W_vendor[:, :_F].T
    wvb_t = W_vendor[:, _F:].T
    wra_t = W_refine[:, :_HH].T
    wrb_t = W_refine[:, _HH:].T
    pc = jnp.stack([b_corr, g_corr, beta_corr])
    pv = jnp.stack([b_vendor, g_vendor, beta_vendor])
    pr = jnp.stack([b_refine, g_refine, beta_refine])
    tail = jnp.zeros((1, _F), f32).at[0, :_H].set(W_out[0]).at[0, _H].set(b_out[0])

    cnt = _sc_hist(cc, cv, ones16, z16)
    hsc, hsv = _tc1(cnt, x, x_lagged, wc_t, wva_t, wvb_t)
    agg1 = _sc_agg1(hsc, hsv, rc, cc, rv, cv, z48)
    hs2a, hs2b = _tc2(agg1, hsc, hsv, cnt, wra_t, wrb_t, pc, pv)
    agg2 = _sc_agg2(hs2a, hs2b, rc, cc, rv, cv, z48)
    out = _tc3(agg2, hs2a, hs2b, cnt, pr, tail)
    return out[:_N, 0]


# speedup vs baseline: 1.0497x; 1.0497x over previous
import functools

import jax
import jax.numpy as jnp
from jax import lax
from jax.experimental import pallas as pl
from jax.experimental.pallas import tpu as pltpu
from jax.experimental.pallas import tpu_sc as plsc

_N, _NPAD, _F, _HH, _H = 10000, 10240, 128, 48, 96
_NC, _NS, _NW, _CH = 2, 16, 32, 128
_C1H = 80
_C1 = 160
_C2 = 316
_C2H = 158
_CW = 1
_EC = _CW * 128
_PADROW = _NPAD - 1
_STR = _NPAD // _NS
_BN = 1280
_HIGH = lax.Precision.HIGHEST

_mesh = plsc.VectorSubcoreMesh(core_axis_name="c", subcore_axis_name="s")
_sc_params = pltpu.CompilerParams(use_tc_tiling_on_sc=False)


def _hist_scatter(ones_v, acc, idx_v, nh):
    @pl.loop(0, nh)
    def _(j):
        pltpu.sync_copy(ones_v, acc.at[idx_v.at[j]], add=True)


@functools.partial(
    pl.kernel,
    out_type=jax.ShapeDtypeStruct((_NC, 2, _NPAD, 16), jnp.float32),
    mesh=_mesh,
    compiler_params=_sc_params,
    scratch_types=[
        pltpu.VMEM_SHARED((_NPAD, 16), jnp.float32),
        pltpu.VMEM_SHARED((_NPAD, 16), jnp.float32),
        pltpu.VMEM((_EC, 16), jnp.float32),
        pltpu.VMEM((_C1H // _CW, _EC), jnp.int32),
    ],
)
def _sc_hist(cc_hbm, cv_hbm, ones_hbm, zeros_hbm, out_hbm,
             acc_c, acc_v, ones_v, idx_v):
    cid = lax.axis_index("c")
    sid = lax.axis_index("s")
    base = sid * _STR
    pltpu.sync_copy(zeros_hbm, acc_c.at[pl.ds(base, _STR)])
    pltpu.sync_copy(zeros_hbm, acc_v.at[pl.ds(base, _STR)])
    pltpu.sync_copy(ones_hbm, ones_v)
    plsc.subcore_barrier()

    nh = _C1H // _CW
    pltpu.sync_copy(cc_hbm.at[sid, pl.ds(cid * nh, nh)], idx_v)
    _hist_scatter(ones_v, acc_c, idx_v, nh)
    pltpu.sync_copy(cv_hbm.at[sid, pl.ds(cid * nh, nh)], idx_v)
    _hist_scatter(ones_v, acc_v, idx_v, nh)

    plsc.subcore_barrier()
    pltpu.sync_copy(acc_c.at[pl.ds(base, _STR)],
                    out_hbm.at[cid, 0, pl.ds(base, _STR)])
    pltpu.sync_copy(acc_v.at[pl.ds(base, _STR)],
                    out_hbm.at[cid, 1, pl.ds(base, _STR)])


def _agg_edges(table, idxr_v, idxc_v, acc, gbs, semg, sems, drain_src, n_chunks):
    nb = len(gbs)
    for b in range(nb - 1):
        pltpu.async_copy(table.at[idxr_v.at[b]], gbs[b], semg[b])

    @pl.loop(0, n_chunks, step=nb)
    def _(j):
        for b in range(nb):
            jj = j + b
            nxt = jj + (nb - 1)
            bb = (b + nb - 1) % nb
            pltpu.make_async_copy(
                table.at[idxr_v.at[jj]], gbs[b], semg[b]).wait()
            pltpu.async_copy(
                gbs[b], acc.at[idxc_v.at[jj]], sems[b], add=True)

            @pl.when(nxt < n_chunks)
            def _():
                @pl.when(nxt >= nb)
                def _():
                    pltpu.make_async_copy(drain_src, gbs[bb], sems[bb]).wait()

                pltpu.async_copy(
                    table.at[idxr_v.at[nxt]], gbs[bb], semg[bb])

    for b in range(nb):
        pltpu.make_async_copy(drain_src, gbs[b], sems[b]).wait()


_AGG_SCRATCH = [
    pltpu.VMEM_SHARED((_NPAD, _HH), jnp.float32),
    pltpu.VMEM_SHARED((_NPAD, _HH), jnp.float32),
    pltpu.VMEM((_C1 // _CW, _EC), jnp.int32),
    pltpu.VMEM((_C1 // _CW, _EC), jnp.int32),
    pltpu.VMEM((_EC, _HH), jnp.float32),
    pltpu.VMEM((_EC, _HH), jnp.float32),
    pltpu.VMEM((_EC, _HH), jnp.float32),
    pltpu.VMEM((_EC, _HH), jnp.float32),
    pltpu.SemaphoreType.DMA,
    pltpu.SemaphoreType.DMA,
    pltpu.SemaphoreType.DMA,
    pltpu.SemaphoreType.DMA,
    pltpu.SemaphoreType.DMA,
    pltpu.SemaphoreType.DMA,
    pltpu.SemaphoreType.DMA,
    pltpu.SemaphoreType.DMA,
]


@functools.partial(
    pl.kernel,
    out_type=jax.ShapeDtypeStruct((_NC, _NPAD, _HH), jnp.float32),
    mesh=_mesh,
    compiler_params=_sc_params,
    scratch_types=_AGG_SCRATCH,
)
def _sc_agg1(hsc_hbm, hsv_hbm, rc_hbm, cc_hbm, rv_hbm, cv_hbm, zeros_hbm,
             out_hbm, acc, table, idxr_v, idxc_v,
             gb0, gb1, gb2, gb3, sg0, sg1, sg2, sg3, ss0, ss1, ss2, ss3):
    cid = lax.axis_index("c")
    sid = lax.axis_index("s")
    base = sid * _STR
    pltpu.sync_copy(zeros_hbm, acc.at[pl.ds(base, _STR)])

    @pl.when(cid == 0)
    def _():
        pltpu.sync_copy(hsc_hbm.at[pl.ds(base, _STR)], table.at[pl.ds(base, _STR)])
        pltpu.sync_copy(rc_hbm.at[sid], idxr_v)
        pltpu.sync_copy(cc_hbm.at[sid], idxc_v)

    @pl.when(cid == 1)
    def _():
        pltpu.sync_copy(hsv_hbm.at[pl.ds(base, _STR)], table.at[pl.ds(base, _STR)])
        pltpu.sync_copy(rv_hbm.at[sid], idxr_v)
        pltpu.sync_copy(cv_hbm.at[sid], idxc_v)

    plsc.subcore_barrier()
    _agg_edges(table, idxr_v, idxc_v, acc, (gb0, gb1, gb2, gb3),
               (sg0, sg1, sg2, sg3), (ss0, ss1, ss2, ss3),
               zeros_hbm.at[pl.ds(0, _EC)], _C1 // _CW)
    plsc.subcore_barrier()
    pltpu.sync_copy(acc.at[pl.ds(base, _STR)],
                    out_hbm.at[cid, pl.ds(base, _STR)])


@functools.partial(
    pl.kernel,
    out_type=jax.ShapeDtypeStruct((_NC, _NPAD, _HH), jnp.float32),
    mesh=_mesh,
    compiler_params=_sc_params,
    scratch_types=_AGG_SCRATCH,
)
def _sc_agg2(hs2a_hbm, hs2b_hbm, rc_hbm, cc_hbm, rv_hbm, cv_hbm, zeros_hbm,
             out_hbm, acc, table, idxr_v, idxc_v,
             gb0, gb1, gb2, gb3, sg0, sg1, sg2, sg3, ss0, ss1, ss2, ss3):
    cid = lax.axis_index("c")
    sid = lax.axis_index("s")
    base = sid * _STR
    pltpu.sync_copy(zeros_hbm, acc.at[pl.ds(base, _STR)])

    @pl.when(cid == 0)
    def _():
        pltpu.sync_copy(hs2a_hbm.at[pl.ds(base, _STR)], table.at[pl.ds(base, _STR)])

    @pl.when(cid == 1)
    def _():
        pltpu.sync_copy(hs2b_hbm.at[pl.ds(base, _STR)], table.at[pl.ds(base, _STR)])

    plsc.subcore_barrier()

    for r_hbm, c_hbm in ((rc_hbm, cc_hbm), (rv_hbm, cv_hbm)):
        pltpu.sync_copy(r_hbm.at[sid], idxr_v)
        pltpu.sync_copy(c_hbm.at[sid], idxc_v)
        _agg_edges(table, idxr_v, idxc_v, acc, (gb0, gb1, gb2, gb3),
                   (sg0, sg1, sg2, sg3), (ss0, ss1, ss2, ss3),
                   zeros_hbm.at[pl.ds(0, _EC)], _C1 // _CW)

    plsc.subcore_barrier()
    pltpu.sync_copy(acc.at[pl.ds(base, _STR)],
                    out_hbm.at[cid, pl.ds(base, _STR)])


def _ln_elu(v, g, b):
    m = jnp.mean(v, axis=-1, keepdims=True)
    var = jnp.mean((v - m) ** 2, axis=-1, keepdims=True)
    u = (v - m) / jnp.sqrt(var + 1e-5) * g + b
    return jnp.where(u > 0, u, jnp.exp(jnp.minimum(u, 0.0)) - 1.0)


def _deg_scales(cnt):
    cntc = cnt[0, 0, :, 0] + cnt[1, 0, :, 0]
    cntv = cnt[0, 1, :, 0] + cnt[1, 1, :, 0]
    disc = lax.rsqrt(cntc + 1.0)
    disv = lax.rsqrt(cntv + 1.0)
    dis2 = lax.rsqrt(cntc + cntv + 1.0)
    return disc, disv, dis2


def _tc1_body(cnt_ref, x_ref, xl_ref, wc_ref, wva_ref, wvb_ref,
              hsc_ref, hsv_ref):
    disc, disv, _ = _deg_scales(cnt_ref[...])
    xb = x_ref[...]
    hc = jnp.dot(xb, wc_ref[...], precision=_HIGH)
    hv = (jnp.dot(xb, wva_ref[...], precision=_HIGH)
          + jnp.dot(xl_ref[...], wvb_ref[...], precision=_HIGH))
    row = _BN * pl.program_id(0) + lax.broadcasted_iota(jnp.int32, (_BN, 1), 0)
    live = row < _N
    hsc_ref[...] = jnp.where(live, hc * disc[:, None], 0.0)
    hsv_ref[...] = jnp.where(live, hv * disv[:, None], 0.0)


def _tc2_body(agg_ref, hsc_ref, hsv_ref, cnt_ref, wra_ref, wrb_ref,
              pc_ref, pv_ref, hs2a_ref, hs2b_ref):
    disc, disv, dis2 = _deg_scales(cnt_ref[...])
    pc = pc_ref[...]
    pv = pv_ref[...]
    oc = disc[:, None] * (agg_ref[0] + hsc_ref[...]) + pc[0]
    ov = disv[:, None] * (agg_ref[1] + hsv_ref[...]) + pv[0]
    uc = _ln_elu(oc, pc[1], pc[2])
    uv = _ln_elu(ov, pv[1], pv[2])
    h2 = (jnp.dot(uc, wra_ref[...], precision=_HIGH)
          + jnp.dot(uv, wrb_ref[...], precision=_HIGH))
    hs2 = h2 * dis2[:, None]
    hs2a_ref[...] = hs2[:, :_HH]
    hs2b_ref[...] = hs2[:, _HH:]


def _tc3_body(agg2_ref, hs2a_ref, hs2b_ref, cnt_ref, pr_ref, tail_ref, out_ref):
    _, _, dis2 = _deg_scales(cnt_ref[...])
    pr = pr_ref[...]
    agg2 = jnp.concatenate([agg2_ref[0], agg2_ref[1]], axis=1)
    hs2 = jnp.concatenate([hs2a_ref[...], hs2b_ref[...]], axis=1)
    o = dis2[:, None] * (agg2 + hs2) + pr[0]
    u = _ln_elu(o, pr[1], pr[2])
    tail = tail_ref[...]
    wo = tail[0, :_H]
    bo = tail[0, _H]
    out_ref[...] = (jnp.sum(u * wo[None, :], axis=1) + bo)[:, None]


def _full(shape):
    return pl.BlockSpec(shape, lambda i: tuple(0 for _ in shape))


def _rows(w):
    return pl.BlockSpec((_BN, w), lambda i: (i, 0))


_GRID = (_NPAD // _BN,)
_CNT_SPEC = pl.BlockSpec((_NC, 2, _BN, 16), lambda i: (0, 0, i, 0))

_tc1 = pl.pallas_call(
    _tc1_body,
    grid=_GRID,
    in_specs=[_CNT_SPEC, _rows(_F), _rows(_F),
              _full((_F, _HH)), _full((_F, _HH)), _full((_F, _HH))],
    out_specs=(_rows(_HH), _rows(_HH)),
    out_shape=(jax.ShapeDtypeStruct((_NPAD, _HH), jnp.float32),
               jax.ShapeDtypeStruct((_NPAD, _HH), jnp.float32)),
)

_tc2 = pl.pallas_call(
    _tc2_body,
    grid=_GRID,
    in_specs=[pl.BlockSpec((_NC, _BN, _HH), lambda i: (0, i, 0)),
              _rows(_HH), _rows(_HH), _CNT_SPEC,
              _full((_HH, _H)), _full((_HH, _H)),
              _full((3, _HH)), _full((3, _HH))],
    out_specs=(_rows(_HH), _rows(_HH)),
    out_shape=(jax.ShapeDtypeStruct((_NPAD, _HH), jnp.float32),
               jax.ShapeDtypeStruct((_NPAD, _HH), jnp.float32)),
)

_tc3 = pl.pallas_call(
    _tc3_body,
    grid=_GRID,
    in_specs=[pl.BlockSpec((_NC, _BN, _HH), lambda i: (0, i, 0)),
              _rows(_HH), _rows(_HH), _CNT_SPEC,
              _full((3, _H)), _full((1, _F))],
    out_specs=pl.BlockSpec((_BN, 1), lambda i: (i, 0)),
    out_shape=jax.ShapeDtypeStruct((_NPAD, 1), jnp.float32),
)


def _prep_idx(idx, n_parts, n_chunks):
    e = idx.shape[0]
    epad = n_parts * n_chunks * _EC
    p = jnp.full((epad,), _PADROW, jnp.int32).at[:e].set(idx)
    return p.reshape(n_parts, n_chunks, _EC)


def kernel(x, edge_index_corr, edge_index_vendor, x_lagged,
           W_corr, b_corr, g_corr, beta_corr,
           W_vendor, b_vendor, g_vendor, beta_vendor,
           W_refine, b_refine, g_refine, beta_refine,
           W_out, b_out):
    f32 = jnp.float32
    rc = _prep_idx(edge_index_corr[0], _NS, _C1 // _CW)
    cc = _prep_idx(edge_index_corr[1], _NS, _C1 // _CW)
    rv = _prep_idx(edge_index_vendor[0], _NS, _C1 // _CW)
    cv = _prep_idx(edge_index_vendor[1], _NS, _C1 // _CW)

    ones16 = jnp.ones((_EC, 16), f32)
    z16 = jnp.zeros((_STR, 16), f32)
    z48 = jnp.zeros((_STR, _HH), f32)

    wc_t = W_corr.T
    wva_t = W_vendor[:, :_F].T
    wvb_t = W_vendor[:, _F:].T
    wra_t = W_refine[:, :_HH].T
    wrb_t = W_refine[:, _HH:].T
    pc = jnp.stack([b_corr, g_corr, beta_corr])
    pv = jnp.stack([b_vendor, g_vendor, beta_vendor])
    pr = jnp.stack([b_refine, g_refine, beta_refine])
    tail = jnp.zeros((1, _F), f32).at[0, :_H].set(W_out[0]).at[0, _H].set(b_out[0])

    cnt = _sc_hist(cc, cv, ones16, z16)
    hsc, hsv = _tc1(cnt, x, x_lagged, wc_t, wva_t, wvb_t)
    agg1 = _sc_agg1(hsc, hsv, rc, cc, rv, cv, z48)
    hs2a, hs2b = _tc2(agg1, hsc, hsv, cnt, wra_t, wrb_t, pc, pv)
    agg2 = _sc_agg2(hs2a, hs2b, rc, cc, rv, cv, z48)
    out = _tc3(agg2, hs2a, hs2b, cnt, pr, tail)
    return out[:_N, 0]


# speedup vs baseline: 1.1391x; 1.0852x over previous
import functools

import jax
import jax.numpy as jnp
from jax import lax
from jax.experimental import pallas as pl
from jax.experimental.pallas import tpu as pltpu
from jax.experimental.pallas import tpu_sc as plsc

_N, _NPAD, _F, _HH, _H = 10000, 10240, 128, 48, 96
_NC, _NS, _NW, _CH = 2, 16, 32, 128
_C1H = 80
_C1 = 160
_C2 = 316
_C2H = 158
_CW = 1
_EC = _CW * 128
_PADROW = _NPAD - 1
_STR = _NPAD // _NS
_BN = 1280
_HIGH = lax.Precision.HIGHEST

_mesh = plsc.VectorSubcoreMesh(core_axis_name="c", subcore_axis_name="s")
_sc_params = pltpu.CompilerParams(use_tc_tiling_on_sc=False)


def _hist_scatter(ones_v, acc, idx_v, nh):
    @pl.loop(0, nh)
    def _(j):
        pltpu.sync_copy(ones_v, acc.at[idx_v.at[j]], add=True)


@functools.partial(
    pl.kernel,
    out_type=jax.ShapeDtypeStruct((_NC, 2, _NPAD, 16), jnp.float32),
    mesh=_mesh,
    compiler_params=_sc_params,
    scratch_types=[
        pltpu.VMEM_SHARED((_NPAD, 16), jnp.float32),
        pltpu.VMEM_SHARED((_NPAD, 16), jnp.float32),
        pltpu.VMEM((_EC, 16), jnp.float32),
        pltpu.VMEM((_C1H // _CW, _EC), jnp.int32),
    ],
)
def _sc_hist(cc_hbm, cv_hbm, ones_hbm, zeros_hbm, out_hbm,
             acc_c, acc_v, ones_v, idx_v):
    cid = lax.axis_index("c")
    sid = lax.axis_index("s")
    base = sid * _STR
    pltpu.sync_copy(zeros_hbm, acc_c.at[pl.ds(base, _STR)])
    pltpu.sync_copy(zeros_hbm, acc_v.at[pl.ds(base, _STR)])
    pltpu.sync_copy(ones_hbm, ones_v)
    plsc.subcore_barrier()

    nh = _C1H // _CW
    pltpu.sync_copy(cc_hbm.at[sid, pl.ds(cid * nh, nh)], idx_v)
    _hist_scatter(ones_v, acc_c, idx_v, nh)
    pltpu.sync_copy(cv_hbm.at[sid, pl.ds(cid * nh, nh)], idx_v)
    _hist_scatter(ones_v, acc_v, idx_v, nh)

    plsc.subcore_barrier()
    pltpu.sync_copy(acc_c.at[pl.ds(base, _STR)],
                    out_hbm.at[cid, 0, pl.ds(base, _STR)])
    pltpu.sync_copy(acc_v.at[pl.ds(base, _STR)],
                    out_hbm.at[cid, 1, pl.ds(base, _STR)])


def _agg_edges(table, idxr_v, idxc_v, acc, gbs, semg, sems, drain_src, n_chunks):
    nb = len(gbs)
    for b in range(nb - 1):
        pltpu.async_copy(table.at[idxr_v.at[b]], gbs[b], semg[b])

    @pl.loop(0, n_chunks, step=nb)
    def _(j):
        for b in range(nb):
            jj = j + b
            nxt = jj + (nb - 1)
            bb = (b + nb - 1) % nb
            pltpu.make_async_copy(
                table.at[idxr_v.at[jj]], gbs[b], semg[b]).wait()
            pltpu.async_copy(
                gbs[b], acc.at[idxc_v.at[jj]], sems[b], add=True)

            @pl.when(nxt < n_chunks)
            def _():
                @pl.when(nxt >= nb)
                def _():
                    pltpu.make_async_copy(drain_src, gbs[bb], sems[bb]).wait()

                pltpu.async_copy(
                    table.at[idxr_v.at[nxt]], gbs[bb], semg[bb])

    for b in range(nb):
        pltpu.make_async_copy(drain_src, gbs[b], sems[b]).wait()


_AGG_SCRATCH = [
    pltpu.VMEM_SHARED((_NPAD, _HH), jnp.float32),
    pltpu.VMEM_SHARED((_NPAD, _HH), jnp.float32),
    pltpu.VMEM((_C1 // _CW, _EC), jnp.int32),
    pltpu.VMEM((_C1 // _CW, _EC), jnp.int32),
    pltpu.VMEM((_EC, _HH), jnp.float32),
    pltpu.VMEM((_EC, _HH), jnp.float32),
    pltpu.VMEM((_EC, _HH), jnp.float32),
    pltpu.VMEM((_EC, _HH), jnp.float32),
    pltpu.SemaphoreType.DMA,
    pltpu.SemaphoreType.DMA,
    pltpu.SemaphoreType.DMA,
    pltpu.SemaphoreType.DMA,
    pltpu.SemaphoreType.DMA,
    pltpu.SemaphoreType.DMA,
    pltpu.SemaphoreType.DMA,
    pltpu.SemaphoreType.DMA,
]


@functools.partial(
    pl.kernel,
    out_type=jax.ShapeDtypeStruct((_NC, _NPAD, _F), jnp.float32),
    mesh=_mesh,
    compiler_params=_sc_params,
    scratch_types=_AGG_SCRATCH,
)
def _sc_agg1(hsc_hbm, hsv_hbm, rc_hbm, cc_hbm, rv_hbm, cv_hbm, zeros_hbm,
             out_hbm, acc, table, idxr_v, idxc_v,
             gb0, gb1, gb2, gb3, sg0, sg1, sg2, sg3, ss0, ss1, ss2, ss3):
    cid = lax.axis_index("c")
    sid = lax.axis_index("s")
    base = sid * _STR
    pltpu.sync_copy(zeros_hbm, acc.at[pl.ds(base, _STR)])

    @pl.when(cid == 0)
    def _():
        pltpu.sync_copy(hsc_hbm.at[pl.ds(base, _STR), pl.ds(0, _HH)],
                        table.at[pl.ds(base, _STR)])
        pltpu.sync_copy(rc_hbm.at[sid], idxr_v)
        pltpu.sync_copy(cc_hbm.at[sid], idxc_v)

    @pl.when(cid == 1)
    def _():
        pltpu.sync_copy(hsv_hbm.at[pl.ds(base, _STR), pl.ds(0, _HH)],
                        table.at[pl.ds(base, _STR)])
        pltpu.sync_copy(rv_hbm.at[sid], idxr_v)
        pltpu.sync_copy(cv_hbm.at[sid], idxc_v)

    plsc.subcore_barrier()
    _agg_edges(table, idxr_v, idxc_v, acc, (gb0, gb1, gb2, gb3),
               (sg0, sg1, sg2, sg3), (ss0, ss1, ss2, ss3),
               zeros_hbm.at[pl.ds(0, _EC)], _C1 // _CW)
    plsc.subcore_barrier()
    pltpu.sync_copy(acc.at[pl.ds(base, _STR)],
                    out_hbm.at[cid, pl.ds(base, _STR), pl.ds(0, _HH)])


@functools.partial(
    pl.kernel,
    out_type=jax.ShapeDtypeStruct((_NC, _NPAD, _F), jnp.float32),
    mesh=_mesh,
    compiler_params=_sc_params,
    scratch_types=_AGG_SCRATCH,
)
def _sc_agg2(hs2a_hbm, hs2b_hbm, rc_hbm, cc_hbm, rv_hbm, cv_hbm, zeros_hbm,
             out_hbm, acc, table, idxr_v, idxc_v,
             gb0, gb1, gb2, gb3, sg0, sg1, sg2, sg3, ss0, ss1, ss2, ss3):
    cid = lax.axis_index("c")
    sid = lax.axis_index("s")
    base = sid * _STR
    pltpu.sync_copy(zeros_hbm, acc.at[pl.ds(base, _STR)])

    @pl.when(cid == 0)
    def _():
        pltpu.sync_copy(hs2a_hbm.at[pl.ds(base, _STR), pl.ds(0, _HH)],
                        table.at[pl.ds(base, _STR)])

    @pl.when(cid == 1)
    def _():
        pltpu.sync_copy(hs2b_hbm.at[pl.ds(base, _STR), pl.ds(0, _HH)],
                        table.at[pl.ds(base, _STR)])

    plsc.subcore_barrier()

    for r_hbm, c_hbm in ((rc_hbm, cc_hbm), (rv_hbm, cv_hbm)):
        pltpu.sync_copy(r_hbm.at[sid], idxr_v)
        pltpu.sync_copy(c_hbm.at[sid], idxc_v)
        _agg_edges(table, idxr_v, idxc_v, acc, (gb0, gb1, gb2, gb3),
                   (sg0, sg1, sg2, sg3), (ss0, ss1, ss2, ss3),
                   zeros_hbm.at[pl.ds(0, _EC)], _C1 // _CW)

    plsc.subcore_barrier()
    pltpu.sync_copy(acc.at[pl.ds(base, _STR)],
                    out_hbm.at[cid, pl.ds(base, _STR), pl.ds(0, _HH)])


def _ln_elu(v, g, b):
    m = jnp.mean(v, axis=-1, keepdims=True)
    var = jnp.mean((v - m) ** 2, axis=-1, keepdims=True)
    u = (v - m) / jnp.sqrt(var + 1e-5) * g + b
    return jnp.where(u > 0, u, jnp.exp(jnp.minimum(u, 0.0)) - 1.0)


def _deg_scales(cnt):
    cntc = cnt[0, 0, :, 0] + cnt[1, 0, :, 0]
    cntv = cnt[0, 1, :, 0] + cnt[1, 1, :, 0]
    disc = lax.rsqrt(cntc + 1.0)
    disv = lax.rsqrt(cntv + 1.0)
    dis2 = lax.rsqrt(cntc + cntv + 1.0)
    return disc, disv, dis2


def _tc1_body(cnt_ref, x_ref, xl_ref, wc_ref, wva_ref, wvb_ref,
              hsc_ref, hsv_ref, dis_ref):
    disc, disv, dis2 = _deg_scales(cnt_ref[...])
    xb = x_ref[...]
    hc = jnp.dot(xb, wc_ref[...], precision=_HIGH)
    hv = (jnp.dot(xb, wva_ref[...], precision=_HIGH)
          + jnp.dot(xl_ref[...], wvb_ref[...], precision=_HIGH))
    row = _BN * pl.program_id(0) + lax.broadcasted_iota(jnp.int32, (_BN, 1), 0)
    live = row < _N
    zpad = jnp.zeros((_BN, _F - _HH), jnp.float32)
    hsc_ref[...] = jnp.concatenate(
        [jnp.where(live, hc * disc[:, None], 0.0), zpad], axis=1)
    hsv_ref[...] = jnp.concatenate(
        [jnp.where(live, hv * disv[:, None], 0.0), zpad], axis=1)
    dis_ref[...] = jnp.stack([disc, disv, dis2], axis=1)


def _tc2_body(agg_ref, hsc_ref, hsv_ref, dis_ref, wra_ref, wrb_ref,
              pc_ref, pv_ref, hs2a_ref, hs2b_ref):
    dis = dis_ref[...]
    disc, disv, dis2 = dis[:, 0], dis[:, 1], dis[:, 2]
    pc = pc_ref[...]
    pv = pv_ref[...]
    oc = disc[:, None] * (agg_ref[0, :, :_HH] + hsc_ref[..., :_HH]) + pc[0]
    ov = disv[:, None] * (agg_ref[1, :, :_HH] + hsv_ref[..., :_HH]) + pv[0]
    uc = _ln_elu(oc, pc[1], pc[2])
    uv = _ln_elu(ov, pv[1], pv[2])
    h2 = (jnp.dot(uc, wra_ref[...], precision=_HIGH)
          + jnp.dot(uv, wrb_ref[...], precision=_HIGH))
    hs2 = h2 * dis2[:, None]
    zpad = jnp.zeros((_BN, _F - _HH), jnp.float32)
    hs2a_ref[...] = jnp.concatenate([hs2[:, :_HH], zpad], axis=1)
    hs2b_ref[...] = jnp.concatenate([hs2[:, _HH:], zpad], axis=1)


def _tc3_body(agg2_ref, hs2a_ref, hs2b_ref, dis_ref, pr_ref, tail_ref, out_ref):
    dis2 = dis_ref[...][:, 2]
    pr = pr_ref[...]
    agg2 = jnp.concatenate([agg2_ref[0, :, :_HH], agg2_ref[1, :, :_HH]], axis=1)
    hs2 = jnp.concatenate([hs2a_ref[..., :_HH], hs2b_ref[..., :_HH]], axis=1)
    o = dis2[:, None] * (agg2 + hs2) + pr[0]
    u = _ln_elu(o, pr[1], pr[2])
    tail = tail_ref[...]
    wo = tail[0, :_H]
    bo = tail[0, _H]
    out_ref[...] = (jnp.sum(u * wo[None, :], axis=1) + bo)[:, None]


def _full(shape):
    return pl.BlockSpec(shape, lambda i: tuple(0 for _ in shape))


def _rows(w):
    return pl.BlockSpec((_BN, w), lambda i: (i, 0))


_GRID = (_NPAD // _BN,)
_CNT_SPEC = pl.BlockSpec((_NC, 2, _BN, 16), lambda i: (0, 0, i, 0))

_tc1 = pl.pallas_call(
    _tc1_body,
    grid=_GRID,
    in_specs=[_CNT_SPEC, _rows(_F), _rows(_F),
              _full((_F, _HH)), _full((_F, _HH)), _full((_F, _HH))],
    out_specs=(_rows(_F), _rows(_F), _rows(3)),
    out_shape=(jax.ShapeDtypeStruct((_NPAD, _F), jnp.float32),
               jax.ShapeDtypeStruct((_NPAD, _F), jnp.float32),
               jax.ShapeDtypeStruct((_NPAD, 3), jnp.float32)),
)

_tc2 = pl.pallas_call(
    _tc2_body,
    grid=_GRID,
    in_specs=[pl.BlockSpec((_NC, _BN, _F), lambda i: (0, i, 0)),
              _rows(_F), _rows(_F), _rows(3),
              _full((_HH, _H)), _full((_HH, _H)),
              _full((3, _HH)), _full((3, _HH))],
    out_specs=(_rows(_F), _rows(_F)),
    out_shape=(jax.ShapeDtypeStruct((_NPAD, _F), jnp.float32),
               jax.ShapeDtypeStruct((_NPAD, _F), jnp.float32)),
)

_tc3 = pl.pallas_call(
    _tc3_body,
    grid=_GRID,
    in_specs=[pl.BlockSpec((_NC, _BN, _F), lambda i: (0, i, 0)),
              _rows(_F), _rows(_F), _rows(3),
              _full((3, _H)), _full((1, _F))],
    out_specs=pl.BlockSpec((_BN, 1), lambda i: (i, 0)),
    out_shape=jax.ShapeDtypeStruct((_NPAD, 1), jnp.float32),
)


def _prep_idx(idx, n_parts, n_chunks):
    e = idx.shape[0]
    epad = n_parts * n_chunks * _EC
    p = jnp.full((epad,), _PADROW, jnp.int32).at[:e].set(idx)
    return p.reshape(n_parts, n_chunks, _EC)


def kernel(x, edge_index_corr, edge_index_vendor, x_lagged,
           W_corr, b_corr, g_corr, beta_corr,
           W_vendor, b_vendor, g_vendor, beta_vendor,
           W_refine, b_refine, g_refine, beta_refine,
           W_out, b_out):
    f32 = jnp.float32
    rc = _prep_idx(edge_index_corr[0], _NS, _C1 // _CW)
    cc = _prep_idx(edge_index_corr[1], _NS, _C1 // _CW)
    rv = _prep_idx(edge_index_vendor[0], _NS, _C1 // _CW)
    cv = _prep_idx(edge_index_vendor[1], _NS, _C1 // _CW)

    ones16 = jnp.ones((_EC, 16), f32)
    z16 = jnp.zeros((_STR, 16), f32)
    z48 = jnp.zeros((_STR, _HH), f32)

    wc_t = W_corr.T
    wva_t = W_vendor[:, :_F].T
    wvb_t = W_vendor[:, _F:].T
    wra_t = W_refine[:, :_HH].T
    wrb_t = W_refine[:, _HH:].T
    pc = jnp.stack([b_corr, g_corr, beta_corr])
    pv = jnp.stack([b_vendor, g_vendor, beta_vendor])
    pr = jnp.stack([b_refine, g_refine, beta_refine])
    tail = jnp.zeros((1, _F), f32).at[0, :_H].set(W_out[0]).at[0, _H].set(b_out[0])

    cnt = _sc_hist(cc, cv, ones16, z16)
    hsc, hsv, dis = _tc1(cnt, x, x_lagged, wc_t, wva_t, wvb_t)
    agg1 = _sc_agg1(hsc, hsv, rc, cc, rv, cv, z48)
    hs2a, hs2b = _tc2(agg1, hsc, hsv, dis, wra_t, wrb_t, pc, pv)
    agg2 = _sc_agg2(hs2a, hs2b, rc, cc, rv, cv, z48)
    out = _tc3(agg2, hs2a, hs2b, dis, pr, tail)
    return out[:_N, 0]


# speedup vs baseline: 1.1617x; 1.0198x over previous
import functools

import jax
import jax.numpy as jnp
from jax import lax
from jax.experimental import pallas as pl
from jax.experimental.pallas import tpu as pltpu
from jax.experimental.pallas import tpu_sc as plsc

_N, _NPAD, _F, _HH, _H = 10000, 10240, 128, 48, 96
_NC, _NS, _NW, _CH = 2, 16, 32, 128
_C1H = 80
_C1 = 160
_C2 = 316
_C2H = 158
_CW = 1
_EC = _CW * 128
_PADROW = _NPAD - 1
_STR = _NPAD // _NS
_BN = 1280
_HIGH = lax.Precision.HIGHEST

_mesh = plsc.VectorSubcoreMesh(core_axis_name="c", subcore_axis_name="s")
_sc_params = pltpu.CompilerParams(use_tc_tiling_on_sc=False)


def _hist_scatter(ones_v, acc, idx_v, nh):
    @pl.loop(0, nh)
    def _(j):
        pltpu.sync_copy(ones_v, acc.at[idx_v.at[j]], add=True)


@functools.partial(
    pl.kernel,
    out_type=jax.ShapeDtypeStruct((_NC, 2, _NPAD, _F), jnp.float32),
    mesh=_mesh,
    compiler_params=_sc_params,
    scratch_types=[
        pltpu.VMEM_SHARED((_NPAD, 16), jnp.float32),
        pltpu.VMEM_SHARED((_NPAD, 16), jnp.float32),
        pltpu.VMEM((_EC, 16), jnp.float32),
        pltpu.VMEM((_C1H // _CW, _EC), jnp.int32),
    ],
)
def _sc_hist(cc_hbm, cv_hbm, ones_hbm, zeros_hbm, out_hbm,
             acc_c, acc_v, ones_v, idx_v):
    cid = lax.axis_index("c")
    sid = lax.axis_index("s")
    base = sid * _STR
    pltpu.sync_copy(zeros_hbm, acc_c.at[pl.ds(base, _STR)])
    pltpu.sync_copy(zeros_hbm, acc_v.at[pl.ds(base, _STR)])
    pltpu.sync_copy(ones_hbm, ones_v)
    plsc.subcore_barrier()

    nh = _C1H // _CW
    pltpu.sync_copy(cc_hbm.at[sid, pl.ds(cid * nh, nh)], idx_v)
    _hist_scatter(ones_v, acc_c, idx_v, nh)
    pltpu.sync_copy(cv_hbm.at[sid, pl.ds(cid * nh, nh)], idx_v)
    _hist_scatter(ones_v, acc_v, idx_v, nh)

    plsc.subcore_barrier()
    pltpu.sync_copy(acc_c.at[pl.ds(base, _STR)],
                    out_hbm.at[cid, 0, pl.ds(base, _STR), pl.ds(0, 16)])
    pltpu.sync_copy(acc_v.at[pl.ds(base, _STR)],
                    out_hbm.at[cid, 1, pl.ds(base, _STR), pl.ds(0, 16)])


def _agg_edges(table, idxr_v, idxc_v, acc, gbs, semg, sems, drain_src, n_chunks):
    nb = len(gbs)
    for b in range(nb - 1):
        pltpu.async_copy(table.at[idxr_v.at[b]], gbs[b], semg[b])

    @pl.loop(0, n_chunks, step=nb)
    def _(j):
        for b in range(nb):
            jj = j + b
            nxt = jj + (nb - 1)
            bb = (b + nb - 1) % nb
            pltpu.make_async_copy(
                table.at[idxr_v.at[jj]], gbs[b], semg[b]).wait()
            pltpu.async_copy(
                gbs[b], acc.at[idxc_v.at[jj]], sems[b], add=True)

            @pl.when(nxt < n_chunks)
            def _():
                @pl.when(nxt >= nb)
                def _():
                    pltpu.make_async_copy(drain_src, gbs[bb], sems[bb]).wait()

                pltpu.async_copy(
                    table.at[idxr_v.at[nxt]], gbs[bb], semg[bb])

    for b in range(nb):
        pltpu.make_async_copy(drain_src, gbs[b], sems[b]).wait()


_AGG_SCRATCH = [
    pltpu.VMEM_SHARED((_NPAD, _HH), jnp.float32),
    pltpu.VMEM_SHARED((_NPAD, _HH), jnp.float32),
    pltpu.VMEM((_C1 // _CW, _EC), jnp.int32),
    pltpu.VMEM((_C1 // _CW, _EC), jnp.int32),
    pltpu.VMEM((_EC, _HH), jnp.float32),
    pltpu.VMEM((_EC, _HH), jnp.float32),
    pltpu.VMEM((_EC, _HH), jnp.float32),
    pltpu.VMEM((_EC, _HH), jnp.float32),
    pltpu.SemaphoreType.DMA,
    pltpu.SemaphoreType.DMA,
    pltpu.SemaphoreType.DMA,
    pltpu.SemaphoreType.DMA,
    pltpu.SemaphoreType.DMA,
    pltpu.SemaphoreType.DMA,
    pltpu.SemaphoreType.DMA,
    pltpu.SemaphoreType.DMA,
]


@functools.partial(
    pl.kernel,
    out_type=jax.ShapeDtypeStruct((_NC, _NPAD, _F), jnp.float32),
    mesh=_mesh,
    compiler_params=_sc_params,
    scratch_types=_AGG_SCRATCH,
)
def _sc_agg1(hsc_hbm, hsv_hbm, rc_hbm, cc_hbm, rv_hbm, cv_hbm, zeros_hbm,
             out_hbm, acc, table, idxr_v, idxc_v,
             gb0, gb1, gb2, gb3, sg0, sg1, sg2, sg3, ss0, ss1, ss2, ss3):
    cid = lax.axis_index("c")
    sid = lax.axis_index("s")
    base = sid * _STR
    pltpu.sync_copy(zeros_hbm, acc.at[pl.ds(base, _STR)])

    @pl.when(cid == 0)
    def _():
        pltpu.sync_copy(hsc_hbm.at[pl.ds(base, _STR), pl.ds(0, _HH)],
                        table.at[pl.ds(base, _STR)])
        pltpu.sync_copy(rc_hbm.at[sid], idxr_v)
        pltpu.sync_copy(cc_hbm.at[sid], idxc_v)

    @pl.when(cid == 1)
    def _():
        pltpu.sync_copy(hsv_hbm.at[pl.ds(base, _STR), pl.ds(0, _HH)],
                        table.at[pl.ds(base, _STR)])
        pltpu.sync_copy(rv_hbm.at[sid], idxr_v)
        pltpu.sync_copy(cv_hbm.at[sid], idxc_v)

    plsc.subcore_barrier()
    _agg_edges(table, idxr_v, idxc_v, acc, (gb0, gb1, gb2, gb3),
               (sg0, sg1, sg2, sg3), (ss0, ss1, ss2, ss3),
               zeros_hbm.at[pl.ds(0, _EC)], _C1 // _CW)
    plsc.subcore_barrier()
    pltpu.sync_copy(acc.at[pl.ds(base, _STR)],
                    out_hbm.at[cid, pl.ds(base, _STR), pl.ds(0, _HH)])


@functools.partial(
    pl.kernel,
    out_type=jax.ShapeDtypeStruct((_NC, _NPAD, _F), jnp.float32),
    mesh=_mesh,
    compiler_params=_sc_params,
    scratch_types=_AGG_SCRATCH,
)
def _sc_agg2(hs2a_hbm, hs2b_hbm, rc_hbm, cc_hbm, rv_hbm, cv_hbm, zeros_hbm,
             out_hbm, acc, table, idxr_v, idxc_v,
             gb0, gb1, gb2, gb3, sg0, sg1, sg2, sg3, ss0, ss1, ss2, ss3):
    cid = lax.axis_index("c")
    sid = lax.axis_index("s")
    base = sid * _STR
    pltpu.sync_copy(zeros_hbm, acc.at[pl.ds(base, _STR)])

    @pl.when(cid == 0)
    def _():
        pltpu.sync_copy(hs2a_hbm.at[pl.ds(base, _STR), pl.ds(0, _HH)],
                        table.at[pl.ds(base, _STR)])

    @pl.when(cid == 1)
    def _():
        pltpu.sync_copy(hs2b_hbm.at[pl.ds(base, _STR), pl.ds(0, _HH)],
                        table.at[pl.ds(base, _STR)])

    plsc.subcore_barrier()

    for r_hbm, c_hbm in ((rc_hbm, cc_hbm), (rv_hbm, cv_hbm)):
        pltpu.sync_copy(r_hbm.at[sid], idxr_v)
        pltpu.sync_copy(c_hbm.at[sid], idxc_v)
        _agg_edges(table, idxr_v, idxc_v, acc, (gb0, gb1, gb2, gb3),
                   (sg0, sg1, sg2, sg3), (ss0, ss1, ss2, ss3),
                   zeros_hbm.at[pl.ds(0, _EC)], _C1 // _CW)

    plsc.subcore_barrier()
    pltpu.sync_copy(acc.at[pl.ds(base, _STR)],
                    out_hbm.at[cid, pl.ds(base, _STR), pl.ds(0, _HH)])


def _ln_elu(v, g, b):
    m = jnp.mean(v, axis=-1, keepdims=True)
    var = jnp.mean((v - m) ** 2, axis=-1, keepdims=True)
    u = (v - m) / jnp.sqrt(var + 1e-5) * g + b
    return jnp.where(u > 0, u, jnp.exp(jnp.minimum(u, 0.0)) - 1.0)


def _deg_scales(cnt):
    cntc = cnt[0, 0, :, 0] + cnt[1, 0, :, 0]
    cntv = cnt[0, 1, :, 0] + cnt[1, 1, :, 0]
    disc = lax.rsqrt(cntc + 1.0)
    disv = lax.rsqrt(cntv + 1.0)
    dis2 = lax.rsqrt(cntc + cntv + 1.0)
    return disc, disv, dis2


def _tc1_body(cnt_ref, x_ref, xl_ref, wc_ref, wva_ref, wvb_ref,
              hsc_ref, hsv_ref, dis_ref):
    disc, disv, dis2 = _deg_scales(cnt_ref[...])
    xb = x_ref[...]
    hc = jnp.dot(xb, wc_ref[...], precision=_HIGH)
    hv = (jnp.dot(xb, wva_ref[...], precision=_HIGH)
          + jnp.dot(xl_ref[...], wvb_ref[...], precision=_HIGH))
    row = _BN * pl.program_id(0) + lax.broadcasted_iota(jnp.int32, (_BN, 1), 0)
    live = row < _N
    zpad = jnp.zeros((_BN, _F - _HH), jnp.float32)
    hsc_ref[...] = jnp.concatenate(
        [jnp.where(live, hc * disc[:, None], 0.0), zpad], axis=1)
    hsv_ref[...] = jnp.concatenate(
        [jnp.where(live, hv * disv[:, None], 0.0), zpad], axis=1)
    dis_ref[...] = jnp.stack([disc, disv, dis2], axis=1)


def _tc2_body(agg_ref, hsc_ref, hsv_ref, dis_ref, wra_ref, wrb_ref,
              pc_ref, pv_ref, hs2a_ref, hs2b_ref):
    dis = dis_ref[...]
    disc, disv, dis2 = dis[:, 0], dis[:, 1], dis[:, 2]
    pc = pc_ref[...]
    pv = pv_ref[...]
    oc = disc[:, None] * (agg_ref[0, :, :_HH] + hsc_ref[..., :_HH]) + pc[0]
    ov = disv[:, None] * (agg_ref[1, :, :_HH] + hsv_ref[..., :_HH]) + pv[0]
    uc = _ln_elu(oc, pc[1], pc[2])
    uv = _ln_elu(ov, pv[1], pv[2])
    h2 = (jnp.dot(uc, wra_ref[...], precision=_HIGH)
          + jnp.dot(uv, wrb_ref[...], precision=_HIGH))
    hs2 = h2 * dis2[:, None]
    zpad = jnp.zeros((_BN, _F - _HH), jnp.float32)
    hs2a_ref[...] = jnp.concatenate([hs2[:, :_HH], zpad], axis=1)
    hs2b_ref[...] = jnp.concatenate([hs2[:, _HH:], zpad], axis=1)


def _tc3_body(agg2_ref, hs2a_ref, hs2b_ref, dis_ref, pr_ref, tail_ref, out_ref):
    dis2 = dis_ref[...][:, 2]
    pr = pr_ref[...]
    agg2 = jnp.concatenate([agg2_ref[0, :, :_HH], agg2_ref[1, :, :_HH]], axis=1)
    hs2 = jnp.concatenate([hs2a_ref[..., :_HH], hs2b_ref[..., :_HH]], axis=1)
    o = dis2[:, None] * (agg2 + hs2) + pr[0]
    u = _ln_elu(o, pr[1], pr[2])
    tail = tail_ref[...]
    wo = tail[0, :_H]
    bo = tail[0, _H]
    out_ref[...] = (jnp.sum(u * wo[None, :], axis=1) + bo)[:, None]


def _full(shape):
    return pl.BlockSpec(shape, lambda i: tuple(0 for _ in shape))


def _rows(w):
    return pl.BlockSpec((_BN, w), lambda i: (i, 0))


_GRID = (_NPAD // _BN,)
_CNT_SPEC = pl.BlockSpec((_NC, 2, _BN, _F), lambda i: (0, 0, i, 0))

_tc1 = pl.pallas_call(
    _tc1_body,
    grid=_GRID,
    in_specs=[_CNT_SPEC, _rows(_F), _rows(_F),
              _full((_F, _HH)), _full((_F, _HH)), _full((_F, _HH))],
    out_specs=(_rows(_F), _rows(_F), _rows(3)),
    out_shape=(jax.ShapeDtypeStruct((_NPAD, _F), jnp.float32),
               jax.ShapeDtypeStruct((_NPAD, _F), jnp.float32),
               jax.ShapeDtypeStruct((_NPAD, 3), jnp.float32)),
)

_tc2 = pl.pallas_call(
    _tc2_body,
    grid=_GRID,
    in_specs=[pl.BlockSpec((_NC, _BN, _F), lambda i: (0, i, 0)),
              _rows(_F), _rows(_F), _rows(3),
              _full((_HH, _H)), _full((_HH, _H)),
              _full((3, _HH)), _full((3, _HH))],
    out_specs=(_rows(_F), _rows(_F)),
    out_shape=(jax.ShapeDtypeStruct((_NPAD, _F), jnp.float32),
               jax.ShapeDtypeStruct((_NPAD, _F), jnp.float32)),
)

_tc3 = pl.pallas_call(
    _tc3_body,
    grid=_GRID,
    in_specs=[pl.BlockSpec((_NC, _BN, _F), lambda i: (0, i, 0)),
              _rows(_F), _rows(_F), _rows(3),
              _full((3, _H)), _full((1, _F))],
    out_specs=pl.BlockSpec((_BN, 1), lambda i: (i, 0)),
    out_shape=jax.ShapeDtypeStruct((_NPAD, 1), jnp.float32),
)


def _prep_idx(idx, n_parts, n_chunks):
    e = idx.shape[0]
    epad = n_parts * n_chunks * _EC
    p = jnp.full((epad,), _PADROW, jnp.int32).at[:e].set(idx)
    return p.reshape(n_parts, n_chunks, _EC)


def kernel(x, edge_index_corr, edge_index_vendor, x_lagged,
           W_corr, b_corr, g_corr, beta_corr,
           W_vendor, b_vendor, g_vendor, beta_vendor,
           W_refine, b_refine, g_refine, beta_refine,
           W_out, b_out):
    f32 = jnp.float32
    rc = _prep_idx(edge_index_corr[0], _NS, _C1 // _CW)
    cc = _prep_idx(edge_index_corr[1], _NS, _C1 // _CW)
    rv = _prep_idx(edge_index_vendor[0], _NS, _C1 // _CW)
    cv = _prep_idx(edge_index_vendor[1], _NS, _C1 // _CW)

    ones16 = jnp.ones((_EC, 16), f32)
    z16 = jnp.zeros((_STR, 16), f32)
    z48 = jnp.zeros((_STR, _HH), f32)

    wc_t = W_corr.T
    wva_t = W_vendor[:, :_F].T
    wvb_t = W_vendor[:, _F:].T
    wra_t = W_refine[:, :_HH].T
    wrb_t = W_refine[:, _HH:].T
    pc = jnp.stack([b_corr, g_corr, beta_corr])
    pv = jnp.stack([b_vendor, g_vendor, beta_vendor])
    pr = jnp.stack([b_refine, g_refine, beta_refine])
    tail = jnp.zeros((1, _F), f32).at[0, :_H].set(W_out[0]).at[0, _H].set(b_out[0])

    cnt = _sc_hist(cc, cv, ones16, z16)
    hsc, hsv, dis = _tc1(cnt, x, x_lagged, wc_t, wva_t, wvb_t)
    agg1 = _sc_agg1(hsc, hsv, rc, cc, rv, cv, z48)
    hs2a, hs2b = _tc2(agg1, hsc, hsv, dis, wra_t, wrb_t, pc, pv)
    agg2 = _sc_agg2(hs2a, hs2b, rc, cc, rv, cv, z48)
    out = _tc3(agg2, hs2a, hs2b, dis, pr, tail)
    return out[:_N, 0]


# speedup vs baseline: 1.1856x; 1.0206x over previous
import functools

import jax
import jax.numpy as jnp
from jax import lax
from jax.experimental import pallas as pl
from jax.experimental.pallas import tpu as pltpu
from jax.experimental.pallas import tpu_sc as plsc

_N, _NPAD, _F, _HH, _H = 10000, 10240, 128, 48, 96
_NC, _NS, _NW, _CH = 2, 16, 32, 128
_C1H = 80
_C1 = 160
_C2 = 316
_C2H = 158
_CW = 1
_EC = _CW * 128
_PADROW = _NPAD - 1
_STR = _NPAD // _NS
_BN = 1280
_HIGH = lax.Precision.DEFAULT

_mesh = plsc.VectorSubcoreMesh(core_axis_name="c", subcore_axis_name="s")
_sc_params = pltpu.CompilerParams(use_tc_tiling_on_sc=False)


def _hist_scatter(ones_v, acc, idx_v, nh):
    @pl.loop(0, nh)
    def _(j):
        pltpu.sync_copy(ones_v, acc.at[idx_v.at[j]], add=True)


@functools.partial(
    pl.kernel,
    out_type=jax.ShapeDtypeStruct((_NC, 2, _NPAD, _F), jnp.float32),
    mesh=_mesh,
    compiler_params=_sc_params,
    scratch_types=[
        pltpu.VMEM_SHARED((_NPAD, 16), jnp.float32),
        pltpu.VMEM_SHARED((_NPAD, 16), jnp.float32),
        pltpu.VMEM((_EC, 16), jnp.float32),
        pltpu.VMEM((_C1H // _CW, _EC), jnp.int32),
    ],
)
def _sc_hist(cc_hbm, cv_hbm, ones_hbm, zeros_hbm, out_hbm,
             acc_c, acc_v, ones_v, idx_v):
    cid = lax.axis_index("c")
    sid = lax.axis_index("s")
    base = sid * _STR
    pltpu.sync_copy(zeros_hbm, acc_c.at[pl.ds(base, _STR)])
    pltpu.sync_copy(zeros_hbm, acc_v.at[pl.ds(base, _STR)])
    pltpu.sync_copy(ones_hbm, ones_v)
    plsc.subcore_barrier()

    nh = _C1H // _CW
    pltpu.sync_copy(cc_hbm.at[sid, pl.ds(cid * nh, nh)], idx_v)
    _hist_scatter(ones_v, acc_c, idx_v, nh)
    pltpu.sync_copy(cv_hbm.at[sid, pl.ds(cid * nh, nh)], idx_v)
    _hist_scatter(ones_v, acc_v, idx_v, nh)

    plsc.subcore_barrier()
    pltpu.sync_copy(acc_c.at[pl.ds(base, _STR)],
                    out_hbm.at[cid, 0, pl.ds(base, _STR), pl.ds(0, 16)])
    pltpu.sync_copy(acc_v.at[pl.ds(base, _STR)],
                    out_hbm.at[cid, 1, pl.ds(base, _STR), pl.ds(0, 16)])


def _agg_edges(table, idxr_v, idxc_v, acc, gbs, semg, sems, drain_src, n_chunks):
    nb = len(gbs)
    for b in range(nb - 1):
        pltpu.async_copy(table.at[idxr_v.at[b]], gbs[b], semg[b])

    @pl.loop(0, n_chunks, step=nb)
    def _(j):
        for b in range(nb):
            jj = j + b
            nxt = jj + (nb - 1)
            bb = (b + nb - 1) % nb
            pltpu.make_async_copy(
                table.at[idxr_v.at[jj]], gbs[b], semg[b]).wait()
            pltpu.async_copy(
                gbs[b], acc.at[idxc_v.at[jj]], sems[b], add=True)

            @pl.when(nxt < n_chunks)
            def _():
                @pl.when(nxt >= nb)
                def _():
                    pltpu.make_async_copy(drain_src, gbs[bb], sems[bb]).wait()

                pltpu.async_copy(
                    table.at[idxr_v.at[nxt]], gbs[bb], semg[bb])

    for b in range(nb):
        pltpu.make_async_copy(drain_src, gbs[b], sems[b]).wait()


_AGG_SCRATCH = [
    pltpu.VMEM_SHARED((_NPAD, _HH), jnp.float32),
    pltpu.VMEM_SHARED((_NPAD, _HH), jnp.float32),
    pltpu.VMEM((_C1 // _CW, _EC), jnp.int32),
    pltpu.VMEM((_C1 // _CW, _EC), jnp.int32),
    pltpu.VMEM((_EC, _HH), jnp.float32),
    pltpu.VMEM((_EC, _HH), jnp.float32),
    pltpu.VMEM((_EC, _HH), jnp.float32),
    pltpu.VMEM((_EC, _HH), jnp.float32),
    pltpu.SemaphoreType.DMA,
    pltpu.SemaphoreType.DMA,
    pltpu.SemaphoreType.DMA,
    pltpu.SemaphoreType.DMA,
    pltpu.SemaphoreType.DMA,
    pltpu.SemaphoreType.DMA,
    pltpu.SemaphoreType.DMA,
    pltpu.SemaphoreType.DMA,
]


@functools.partial(
    pl.kernel,
    out_type=jax.ShapeDtypeStruct((_NC, _NPAD, _F), jnp.float32),
    mesh=_mesh,
    compiler_params=_sc_params,
    scratch_types=_AGG_SCRATCH,
)
def _sc_agg1(hsc_hbm, hsv_hbm, rc_hbm, cc_hbm, rv_hbm, cv_hbm, zeros_hbm,
             out_hbm, acc, table, idxr_v, idxc_v,
             gb0, gb1, gb2, gb3, sg0, sg1, sg2, sg3, ss0, ss1, ss2, ss3):
    cid = lax.axis_index("c")
    sid = lax.axis_index("s")
    base = sid * _STR
    pltpu.sync_copy(zeros_hbm, acc.at[pl.ds(base, _STR)])

    @pl.when(cid == 0)
    def _():
        pltpu.sync_copy(hsc_hbm.at[pl.ds(base, _STR), pl.ds(0, _HH)],
                        table.at[pl.ds(base, _STR)])
        pltpu.sync_copy(rc_hbm.at[sid], idxr_v)
        pltpu.sync_copy(cc_hbm.at[sid], idxc_v)

    @pl.when(cid == 1)
    def _():
        pltpu.sync_copy(hsv_hbm.at[pl.ds(base, _STR), pl.ds(0, _HH)],
                        table.at[pl.ds(base, _STR)])
        pltpu.sync_copy(rv_hbm.at[sid], idxr_v)
        pltpu.sync_copy(cv_hbm.at[sid], idxc_v)

    plsc.subcore_barrier()
    _agg_edges(table, idxr_v, idxc_v, acc, (gb0, gb1, gb2, gb3),
               (sg0, sg1, sg2, sg3), (ss0, ss1, ss2, ss3),
               zeros_hbm.at[pl.ds(0, _EC)], _C1 // _CW)
    plsc.subcore_barrier()
    pltpu.sync_copy(acc.at[pl.ds(base, _STR)],
                    out_hbm.at[cid, pl.ds(base, _STR), pl.ds(0, _HH)])


@functools.partial(
    pl.kernel,
    out_type=jax.ShapeDtypeStruct((_NC, _NPAD, _F), jnp.float32),
    mesh=_mesh,
    compiler_params=_sc_params,
    scratch_types=_AGG_SCRATCH,
)
def _sc_agg2(hs2a_hbm, hs2b_hbm, rc_hbm, cc_hbm, rv_hbm, cv_hbm, zeros_hbm,
             out_hbm, acc, table, idxr_v, idxc_v,
             gb0, gb1, gb2, gb3, sg0, sg1, sg2, sg3, ss0, ss1, ss2, ss3):
    cid = lax.axis_index("c")
    sid = lax.axis_index("s")
    base = sid * _STR
    pltpu.sync_copy(zeros_hbm, acc.at[pl.ds(base, _STR)])

    @pl.when(cid == 0)
    def _():
        pltpu.sync_copy(hs2a_hbm.at[pl.ds(base, _STR), pl.ds(0, _HH)],
                        table.at[pl.ds(base, _STR)])

    @pl.when(cid == 1)
    def _():
        pltpu.sync_copy(hs2b_hbm.at[pl.ds(base, _STR), pl.ds(0, _HH)],
                        table.at[pl.ds(base, _STR)])

    plsc.subcore_barrier()

    for r_hbm, c_hbm in ((rc_hbm, cc_hbm), (rv_hbm, cv_hbm)):
        pltpu.sync_copy(r_hbm.at[sid], idxr_v)
        pltpu.sync_copy(c_hbm.at[sid], idxc_v)
        _agg_edges(table, idxr_v, idxc_v, acc, (gb0, gb1, gb2, gb3),
                   (sg0, sg1, sg2, sg3), (ss0, ss1, ss2, ss3),
                   zeros_hbm.at[pl.ds(0, _EC)], _C1 // _CW)

    plsc.subcore_barrier()
    pltpu.sync_copy(acc.at[pl.ds(base, _STR)],
                    out_hbm.at[cid, pl.ds(base, _STR), pl.ds(0, _HH)])


def _ln_elu(v, g, b):
    m = jnp.mean(v, axis=-1, keepdims=True)
    var = jnp.mean((v - m) ** 2, axis=-1, keepdims=True)
    u = (v - m) / jnp.sqrt(var + 1e-5) * g + b
    return jnp.where(u > 0, u, jnp.exp(jnp.minimum(u, 0.0)) - 1.0)


def _deg_scales(cnt):
    cntc = cnt[0, 0, :, 0] + cnt[1, 0, :, 0]
    cntv = cnt[0, 1, :, 0] + cnt[1, 1, :, 0]
    disc = lax.rsqrt(cntc + 1.0)
    disv = lax.rsqrt(cntv + 1.0)
    dis2 = lax.rsqrt(cntc + cntv + 1.0)
    return disc, disv, dis2


def _tc1_body(cnt_ref, x_ref, xl_ref, wc_ref, wva_ref, wvb_ref,
              hsc_ref, hsv_ref, dis_ref):
    disc, disv, dis2 = _deg_scales(cnt_ref[...])
    xb = x_ref[...]
    hc = jnp.dot(xb, wc_ref[...], precision=_HIGH)
    hv = (jnp.dot(xb, wva_ref[...], precision=_HIGH)
          + jnp.dot(xl_ref[...], wvb_ref[...], precision=_HIGH))
    row = _BN * pl.program_id(0) + lax.broadcasted_iota(jnp.int32, (_BN, 1), 0)
    live = row < _N
    zpad = jnp.zeros((_BN, _F - _HH), jnp.float32)
    hsc_ref[...] = jnp.concatenate(
        [jnp.where(live, hc * disc[:, None], 0.0), zpad], axis=1)
    hsv_ref[...] = jnp.concatenate(
        [jnp.where(live, hv * disv[:, None], 0.0), zpad], axis=1)
    dis_ref[...] = jnp.stack([disc, disv, dis2], axis=1)


def _tc2_body(agg_ref, hsc_ref, hsv_ref, dis_ref, wra_ref, wrb_ref,
              pc_ref, pv_ref, hs2a_ref, hs2b_ref):
    dis = dis_ref[...]
    disc, disv, dis2 = dis[:, 0], dis[:, 1], dis[:, 2]
    pc = pc_ref[...]
    pv = pv_ref[...]
    oc = disc[:, None] * (agg_ref[0, :, :_HH] + hsc_ref[..., :_HH]) + pc[0]
    ov = disv[:, None] * (agg_ref[1, :, :_HH] + hsv_ref[..., :_HH]) + pv[0]
    uc = _ln_elu(oc, pc[1], pc[2])
    uv = _ln_elu(ov, pv[1], pv[2])
    h2 = (jnp.dot(uc, wra_ref[...], precision=_HIGH)
          + jnp.dot(uv, wrb_ref[...], precision=_HIGH))
    hs2 = h2 * dis2[:, None]
    zpad = jnp.zeros((_BN, _F - _HH), jnp.float32)
    hs2a_ref[...] = jnp.concatenate([hs2[:, :_HH], zpad], axis=1)
    hs2b_ref[...] = jnp.concatenate([hs2[:, _HH:], zpad], axis=1)


def _tc3_body(agg2_ref, hs2a_ref, hs2b_ref, dis_ref, pr_ref, tail_ref, out_ref):
    dis2 = dis_ref[...][:, 2]
    pr = pr_ref[...]
    agg2 = jnp.concatenate([agg2_ref[0, :, :_HH], agg2_ref[1, :, :_HH]], axis=1)
    hs2 = jnp.concatenate([hs2a_ref[..., :_HH], hs2b_ref[..., :_HH]], axis=1)
    o = dis2[:, None] * (agg2 + hs2) + pr[0]
    u = _ln_elu(o, pr[1], pr[2])
    tail = tail_ref[...]
    wo = tail[0, :_H]
    bo = tail[0, _H]
    out_ref[...] = (jnp.sum(u * wo[None, :], axis=1) + bo)[:, None]


def _full(shape):
    return pl.BlockSpec(shape, lambda i: tuple(0 for _ in shape))


def _rows(w):
    return pl.BlockSpec((_BN, w), lambda i: (i, 0))


_GRID = (_NPAD // _BN,)
_CNT_SPEC = pl.BlockSpec((_NC, 2, _BN, _F), lambda i: (0, 0, i, 0))

_tc1 = pl.pallas_call(
    _tc1_body,
    grid=_GRID,
    in_specs=[_CNT_SPEC, _rows(_F), _rows(_F),
              _full((_F, _HH)), _full((_F, _HH)), _full((_F, _HH))],
    out_specs=(_rows(_F), _rows(_F), _rows(3)),
    out_shape=(jax.ShapeDtypeStruct((_NPAD, _F), jnp.float32),
               jax.ShapeDtypeStruct((_NPAD, _F), jnp.float32),
               jax.ShapeDtypeStruct((_NPAD, 3), jnp.float32)),
)

_tc2 = pl.pallas_call(
    _tc2_body,
    grid=_GRID,
    in_specs=[pl.BlockSpec((_NC, _BN, _F), lambda i: (0, i, 0)),
              _rows(_F), _rows(_F), _rows(3),
              _full((_HH, _H)), _full((_HH, _H)),
              _full((3, _HH)), _full((3, _HH))],
    out_specs=(_rows(_F), _rows(_F)),
    out_shape=(jax.ShapeDtypeStruct((_NPAD, _F), jnp.float32),
               jax.ShapeDtypeStruct((_NPAD, _F), jnp.float32)),
)

_tc3 = pl.pallas_call(
    _tc3_body,
    grid=_GRID,
    in_specs=[pl.BlockSpec((_NC, _BN, _F), lambda i: (0, i, 0)),
              _rows(_F), _rows(_F), _rows(3),
              _full((3, _H)), _full((1, _F))],
    out_specs=pl.BlockSpec((_BN, 1), lambda i: (i, 0)),
    out_shape=jax.ShapeDtypeStruct((_NPAD, 1), jnp.float32),
)


def _prep_idx(idx, n_parts, n_chunks):
    e = idx.shape[0]
    epad = n_parts * n_chunks * _EC
    p = jnp.full((epad,), _PADROW, jnp.int32).at[:e].set(idx)
    return p.reshape(n_parts, n_chunks, _EC)


def kernel(x, edge_index_corr, edge_index_vendor, x_lagged,
           W_corr, b_corr, g_corr, beta_corr,
           W_vendor, b_vendor, g_vendor, beta_vendor,
           W_refine, b_refine, g_refine, beta_refine,
           W_out, b_out):
    f32 = jnp.float32
    rc = _prep_idx(edge_index_corr[0], _NS, _C1 // _CW)
    cc = _prep_idx(edge_index_corr[1], _NS, _C1 // _CW)
    rv = _prep_idx(edge_index_vendor[0], _NS, _C1 // _CW)
    cv = _prep_idx(edge_index_vendor[1], _NS, _C1 // _CW)

    ones16 = jnp.ones((_EC, 16), f32)
    z16 = jnp.zeros((_STR, 16), f32)
    z48 = jnp.zeros((_STR, _HH), f32)

    wc_t = W_corr.T
    wva_t = W_vendor[:, :_F].T
    wvb_t = W_vendor[:, _F:].T
    wra_t = W_refine[:, :_HH].T
    wrb_t = W_refine[:, _HH:].T
    pc = jnp.stack([b_corr, g_corr, beta_corr])
    pv = jnp.stack([b_vendor, g_vendor, beta_vendor])
    pr = jnp.stack([b_refine, g_refine, beta_refine])
    tail = jnp.zeros((1, _F), f32).at[0, :_H].set(W_out[0]).at[0, _H].set(b_out[0])

    cnt = _sc_hist(cc, cv, ones16, z16)
    hsc, hsv, dis = _tc1(cnt, x, x_lagged, wc_t, wva_t, wvb_t)
    agg1 = _sc_agg1(hsc, hsv, rc, cc, rv, cv, z48)
    hs2a, hs2b = _tc2(agg1, hsc, hsv, dis, wra_t, wrb_t, pc, pv)
    agg2 = _sc_agg2(hs2a, hs2b, rc, cc, rv, cv, z48)
    out = _tc3(agg2, hs2a, hs2b, dis, pr, tail)
    return out[:_N, 0]
